# baseline (speedup 1.0000x reference)
import jax
import jax.numpy as jnp
from jax.experimental import pallas as pl

_ROWS = 8
_N = 32768
_CHUNK = 128
_NCHUNKS = _N // _CHUNK
_GROUPS = 8
_NEG = float("-inf")


def _exact_top5_mask(x):
    col = jax.lax.broadcasted_iota(jnp.int32, x.shape, 1)
    work = x
    for _ in range(5):
        m = jnp.max(work, axis=1, keepdims=True)
        eq = work == m
        first_col = jnp.min(jnp.where(eq, col, _N), axis=1, keepdims=True)
        work = jnp.where(col == first_col, _NEG, work)
    return work == _NEG


def _insert(acc, v):
    t1, t2, t3, t4, t5 = acc
    m1 = jnp.maximum(t1, v)
    r1 = jnp.minimum(t1, v)
    m2 = jnp.maximum(t2, r1)
    r2 = jnp.minimum(t2, r1)
    m3 = jnp.maximum(t3, r2)
    r3 = jnp.minimum(t3, r2)
    m4 = jnp.maximum(t4, r3)
    r4 = jnp.minimum(t4, r3)
    m5 = jnp.maximum(t5, r4)
    return (m1, m2, m3, m4, m5)


def _fold(a, b):
    return tuple(jnp.maximum(a[i], b[4 - i]) for i in range(5))


def _top5_round_body(x_ref, o_ref):
    x = x_ref[...]

    per_group = _NCHUNKS // _GROUPS
    accs = [tuple(jnp.full((_ROWS, _CHUNK), _NEG) for _ in range(5))
            for _ in range(_GROUPS)]
    for j in range(per_group):
        for g in range(_GROUPS):
            k = g * per_group + j
            accs[g] = _insert(accs[g], x[:, k * _CHUNK:(k + 1) * _CHUNK])

    folded = [_fold(accs[2 * i], accs[2 * i + 1]) for i in range(_GROUPS // 2)]

    cand = jnp.concatenate([t for f in folded for t in f], axis=1)
    ccol = jax.lax.broadcasted_iota(jnp.int32, cand.shape, 1)
    vals = []
    for _ in range(5):
        m = jnp.max(cand, axis=1, keepdims=True)
        vals.append(m)
        first_col = jnp.min(
            jnp.where(cand == m, ccol, cand.shape[1]), axis=1, keepdims=True)
        cand = jnp.where(ccol == first_col, _NEG, cand)
    v4, v5 = vals[3], vals[4]

    fast = jnp.all(v4 > v5)

    @pl.when(fast)
    def _():
        col = jax.lax.broadcasted_iota(jnp.int32, x.shape, 1)
        first_eq = jnp.min(jnp.where(x == v5, col, _N), axis=1, keepdims=True)
        keep = (x > v5) | (col == first_eq)
        o_ref[...] = jnp.where(keep, jnp.round(x), 0.0)

    @pl.when(jnp.logical_not(fast))
    def _():
        o_ref[...] = jnp.where(_exact_top5_mask(x), jnp.round(x), 0.0)


def kernel(x):
    rows, n = x.shape
    grid = (rows // _ROWS,)
    return pl.pallas_call(
        _top5_round_body,
        grid=grid,
        in_specs=[pl.BlockSpec((_ROWS, n), lambda i: (i, 0))],
        out_specs=pl.BlockSpec((_ROWS, n), lambda i: (i, 0)),
        out_shape=jax.ShapeDtypeStruct(x.shape, x.dtype),
    )(x)


# speedup vs baseline: 1.0898x; 1.0898x over previous
import jax
import jax.numpy as jnp
from jax.experimental import pallas as pl
from jax.experimental.pallas import tpu as pltpu

_ROWS = 8
_N = 32768
_CHUNK = 128
_NCHUNKS = _N // _CHUNK
_GROUPS = 4
_NEG = float("-inf")


def _insert(acc, v):
    t1, t2, t3, t4, t5 = acc
    m1 = jnp.maximum(t1, v)
    r1 = jnp.minimum(t1, v)
    m2 = jnp.maximum(t2, r1)
    r2 = jnp.minimum(t2, r1)
    m3 = jnp.maximum(t3, r2)
    r3 = jnp.minimum(t3, r2)
    m4 = jnp.maximum(t4, r3)
    r4 = jnp.minimum(t4, r3)
    m5 = jnp.maximum(t5, r4)
    return (m1, m2, m3, m4, m5)


def _fold(a, b):
    return tuple(jnp.maximum(a[i], b[4 - i]) for i in range(5))


def _chunk(ref, k):
    return ref[:, k * _CHUNK:(k + 1) * _CHUNK]


def _top5_round_body(x_ref, o_ref, scratch_ref):
    iota = jax.lax.broadcasted_iota(jnp.int32, (_ROWS, _CHUNK), 1)

    per_group = _NCHUNKS // _GROUPS
    accs = [tuple(jnp.full((_ROWS, _CHUNK), _NEG) for _ in range(5))
            for _ in range(_GROUPS)]
    for j in range(per_group):
        for g in range(_GROUPS):
            accs[g] = _insert(accs[g], _chunk(x_ref, g * per_group + j))

    folded = [_fold(accs[2 * i], accs[2 * i + 1]) for i in range(_GROUPS // 2)]

    cand = jnp.concatenate([t for f in folded for t in f], axis=1)
    ccol = jax.lax.broadcasted_iota(jnp.int32, cand.shape, 1)
    vals = []
    for _ in range(5):
        m = jnp.max(cand, axis=1, keepdims=True)
        vals.append(m)
        first_col = jnp.min(
            jnp.where(cand == m, ccol, cand.shape[1]), axis=1, keepdims=True)
        cand = jnp.where(ccol == first_col, _NEG, cand)
    v4, v5 = vals[3], vals[4]

    fast = jnp.all(v4 > v5)

    @pl.when(fast)
    def _():
        parts = []
        for g in range(_GROUPS):
            m = jnp.full((_ROWS, _CHUNK), _N, jnp.int32)
            for j in range(per_group):
                k = g * per_group + j
                v = _chunk(x_ref, k)
                m = jnp.minimum(m, jnp.where(v == v5, iota + k * _CHUNK, _N))
            parts.append(m)
        first_eq = jnp.min(jnp.concatenate(parts, axis=1), axis=1,
                           keepdims=True)
        for k in range(_NCHUNKS):
            v = _chunk(x_ref, k)
            keep = (v > v5) | (iota + k * _CHUNK == first_eq)
            o_ref[:, k * _CHUNK:(k + 1) * _CHUNK] = jnp.where(
                keep, jnp.round(v), 0.0)

    @pl.when(jnp.logical_not(fast))
    def _():
        for k in range(_NCHUNKS):
            scratch_ref[:, k * _CHUNK:(k + 1) * _CHUNK] = _chunk(x_ref, k)
        for _ in range(5):
            m = jnp.full((_ROWS, _CHUNK), _NEG)
            for k in range(_NCHUNKS):
                m = jnp.maximum(m, _chunk(scratch_ref, k))
            m = jnp.max(m, axis=1, keepdims=True)
            fc = jnp.full((_ROWS, _CHUNK), _N, jnp.int32)
            for k in range(_NCHUNKS):
                fc = jnp.minimum(fc, jnp.where(
                    _chunk(scratch_ref, k) == m, iota + k * _CHUNK, _N))
            fc = jnp.min(fc, axis=1, keepdims=True)
            for k in range(_NCHUNKS):
                w = _chunk(scratch_ref, k)
                scratch_ref[:, k * _CHUNK:(k + 1) * _CHUNK] = jnp.where(
                    iota + k * _CHUNK == fc, _NEG, w)
        for k in range(_NCHUNKS):
            v = _chunk(x_ref, k)
            sel = _chunk(scratch_ref, k) == _NEG
            o_ref[:, k * _CHUNK:(k + 1) * _CHUNK] = jnp.where(
                sel, jnp.round(v), 0.0)


def kernel(x):
    rows, n = x.shape
    grid = (rows // _ROWS,)
    return pl.pallas_call(
        _top5_round_body,
        grid=grid,
        in_specs=[pl.BlockSpec((_ROWS, n), lambda i: (i, 0))],
        out_specs=pl.BlockSpec((_ROWS, n), lambda i: (i, 0)),
        out_shape=jax.ShapeDtypeStruct(x.shape, x.dtype),
        scratch_shapes=[pltpu.VMEM((_ROWS, _N), jnp.float32)],
    )(x)


# speedup vs baseline: 1.1919x; 1.0936x over previous
import jax
import jax.numpy as jnp
from jax.experimental import pallas as pl
from jax.experimental.pallas import tpu as pltpu

_ROWS = 8
_N = 32768
_CHUNK = 128
_NCHUNKS = _N // _CHUNK
_GROUPS = 4
_NEG = float("-inf")


def _insert(acc, v):
    t1, t2, t3, t4, t5 = acc
    m1 = jnp.maximum(t1, v)
    r1 = jnp.minimum(t1, v)
    m2 = jnp.maximum(t2, r1)
    r2 = jnp.minimum(t2, r1)
    m3 = jnp.maximum(t3, r2)
    r3 = jnp.minimum(t3, r2)
    m4 = jnp.maximum(t4, r3)
    r4 = jnp.minimum(t4, r3)
    m5 = jnp.maximum(t5, r4)
    return (m1, m2, m3, m4, m5)


def _fold(a, b):
    return tuple(jnp.maximum(a[i], b[4 - i]) for i in range(5))


def _chunk(ref, k):
    return ref[:, k * _CHUNK:(k + 1) * _CHUNK]


def _top5_round_body(x_ref, o_ref, scratch_ref):
    iota = jax.lax.broadcasted_iota(jnp.int32, (_ROWS, _CHUNK), 1)

    per_group = _NCHUNKS // _GROUPS
    accs = [tuple(jnp.full((_ROWS, _CHUNK), _NEG) for _ in range(5))
            for _ in range(_GROUPS)]
    for j in range(per_group):
        for g in range(_GROUPS):
            accs[g] = _insert(accs[g], _chunk(x_ref, g * per_group + j))

    folded = [_fold(accs[2 * i], accs[2 * i + 1]) for i in range(_GROUPS // 2)]

    cand = jnp.concatenate([t for f in folded for t in f], axis=1)
    ccol = jax.lax.broadcasted_iota(jnp.int32, cand.shape, 1)
    vals = []
    for _ in range(5):
        m = jnp.max(cand, axis=1, keepdims=True)
        vals.append(m)
        first_col = jnp.min(
            jnp.where(cand == m, ccol, cand.shape[1]), axis=1, keepdims=True)
        cand = jnp.where(ccol == first_col, _NEG, cand)
    v4, v5 = vals[3], vals[4]

    distinct = jnp.all(v4 > v5)

    eq_cnt = jnp.zeros((_ROWS, _CHUNK), jnp.float32)
    for g in range(_GROUPS):
        for t in accs[g]:
            eq_cnt = eq_cnt + jnp.where(t == v5, 1.0, 0.0)
    unique = jnp.sum(eq_cnt, axis=1, keepdims=True) == 1.0
    rank5_hit = jnp.zeros((_ROWS, _CHUNK), jnp.bool_)
    for g in range(_GROUPS):
        rank5_hit = rank5_hit | (accs[g][4] == v5)
    no_full_group = jnp.logical_not(jnp.any(rank5_hit, axis=1, keepdims=True))

    fast = distinct & jnp.all(unique & no_full_group)
    medium = distinct & jnp.logical_not(fast)

    @pl.when(fast)
    def _():
        for k in range(_NCHUNKS):
            v = _chunk(x_ref, k)
            o_ref[:, k * _CHUNK:(k + 1) * _CHUNK] = jnp.where(
                v >= v5, jnp.round(v), 0.0)

    @pl.when(medium)
    def _():
        parts = []
        for g in range(_GROUPS):
            m = jnp.full((_ROWS, _CHUNK), _N, jnp.int32)
            for j in range(per_group):
                k = g * per_group + j
                v = _chunk(x_ref, k)
                m = jnp.minimum(m, jnp.where(v == v5, iota + k * _CHUNK, _N))
            parts.append(m)
        first_eq = jnp.min(jnp.concatenate(parts, axis=1), axis=1,
                           keepdims=True)
        for k in range(_NCHUNKS):
            v = _chunk(x_ref, k)
            keep = (v > v5) | (iota + k * _CHUNK == first_eq)
            o_ref[:, k * _CHUNK:(k + 1) * _CHUNK] = jnp.where(
                keep, jnp.round(v), 0.0)

    @pl.when(jnp.logical_not(distinct))
    def _():
        for k in range(_NCHUNKS):
            scratch_ref[:, k * _CHUNK:(k + 1) * _CHUNK] = _chunk(x_ref, k)
        for _ in range(5):
            m = jnp.full((_ROWS, _CHUNK), _NEG)
            for k in range(_NCHUNKS):
                m = jnp.maximum(m, _chunk(scratch_ref, k))
            m = jnp.max(m, axis=1, keepdims=True)
            fc = jnp.full((_ROWS, _CHUNK), _N, jnp.int32)
            for k in range(_NCHUNKS):
                fc = jnp.minimum(fc, jnp.where(
                    _chunk(scratch_ref, k) == m, iota + k * _CHUNK, _N))
            fc = jnp.min(fc, axis=1, keepdims=True)
            for k in range(_NCHUNKS):
                w = _chunk(scratch_ref, k)
                scratch_ref[:, k * _CHUNK:(k + 1) * _CHUNK] = jnp.where(
                    iota + k * _CHUNK == fc, _NEG, w)
        for k in range(_NCHUNKS):
            v = _chunk(x_ref, k)
            sel = _chunk(scratch_ref, k) == _NEG
            o_ref[:, k * _CHUNK:(k + 1) * _CHUNK] = jnp.where(
                sel, jnp.round(v), 0.0)


def kernel(x):
    rows, n = x.shape
    grid = (rows // _ROWS,)
    return pl.pallas_call(
        _top5_round_body,
        grid=grid,
        in_specs=[pl.BlockSpec((_ROWS, n), lambda i: (i, 0))],
        out_specs=pl.BlockSpec((_ROWS, n), lambda i: (i, 0)),
        out_shape=jax.ShapeDtypeStruct(x.shape, x.dtype),
        scratch_shapes=[pltpu.VMEM((_ROWS, _N), jnp.float32)],
    )(x)
